# baseline (speedup 1.0000x reference)
import jax
import jax.numpy as jnp
from jax.experimental import pallas as pl
from jax.experimental.pallas import tpu as pltpu

_C = 3
_IMG = 64
_H = 512
_W = 512
_IPC = 4
_CPC = _IMG // _IPC
_NCHUNKS = _C * _CPC
_NBUF = 12
_D = 6


def _body(keep_ref, in_hbm, out_hbm, bufs, zbuf, rsem, wsem):
    zbuf[...] = jnp.zeros_like(zbuf)

    def in_chunk(i):
        c, r = divmod(i, _CPC)
        return in_hbm.at[c, pl.ds(r * _IPC, _IPC)]

    def out_chunk(i):
        c, r = divmod(i, _CPC)
        return out_hbm.at[c, pl.ds(r * _IPC, _IPC)]

    def start_read(i):
        b = i % _NBUF
        kc = keep_ref[i // _CPC]

        @pl.when(kc > 0)
        def _():
            pltpu.make_async_copy(in_chunk(i), bufs.at[b], rsem.at[b]).start()

    def start_write(p):
        b = p % _NBUF
        kc = keep_ref[p // _CPC]

        @pl.when(kc > 0)
        def _():
            pltpu.make_async_copy(in_chunk(p), bufs.at[b], rsem.at[b]).wait()
            pltpu.make_async_copy(bufs.at[b], out_chunk(p), wsem.at[b]).start()

        @pl.when(kc == 0)
        def _():
            pltpu.make_async_copy(zbuf, out_chunk(p), wsem.at[b]).start()

    for i in range(_NCHUNKS + _D):
        if i < _NCHUNKS:
            if i >= _NBUF:
                b = i % _NBUF
                pltpu.make_async_copy(
                    bufs.at[b], out_chunk(i - _NBUF), wsem.at[b]
                ).wait()
            start_read(i)
        if i >= _D:
            start_write(i - _D)

    for p in range(_NCHUNKS - _NBUF, _NCHUNKS):
        b = p % _NBUF
        pltpu.make_async_copy(bufs.at[b], out_chunk(p), wsem.at[b]).wait()


def kernel(tensor, skip_prob):
    u = jax.random.uniform(jax.random.key(42), (3,), dtype=jnp.float32)
    keep = (u > skip_prob).astype(jnp.int32)
    return pl.pallas_call(
        _body,
        in_specs=[
            pl.BlockSpec(memory_space=pltpu.SMEM),
            pl.BlockSpec(memory_space=pl.ANY),
        ],
        out_specs=pl.BlockSpec(memory_space=pl.ANY),
        out_shape=jax.ShapeDtypeStruct((_C, _IMG, _H, _W), jnp.float32),
        scratch_shapes=[
            pltpu.VMEM((_NBUF, _IPC, _H, _W), jnp.float32),
            pltpu.VMEM((_IPC, _H, _W), jnp.float32),
            pltpu.SemaphoreType.DMA((_NBUF,)),
            pltpu.SemaphoreType.DMA((_NBUF,)),
        ],
    )(keep, tensor)
